# Initial kernel scaffold; baseline (speedup 1.0000x reference)
import jax
import jax.numpy as jnp
from jax.experimental import pallas as pl


def kernel(phar_prompt, mol_repr, protein_batch, proj_w1, proj_b1, proj_w2, proj_b2, emb, mol_gamma, mol_beta, prot_gamma, prot_beta, conv_w, conv_b, wa, ba, wout_w, wout_b, wint_w, wint_b):
    raise NotImplementedError("write your pallas kernel here")


# speedup vs baseline: 49.0400x; 49.0400x over previous
import math

import jax
import jax.numpy as jnp
import numpy as np
from jax import lax
from jax.experimental import pallas as pl
from jax.experimental.pallas import tpu as pltpu

SEQ = 16
DIM = 32
NQ = 3
NWORD = 32
WIN = 2
KTAP = 2 * WIN + 1
LCNN = 3
LOUT = 3
LN_EPS = 1e-5


def _layernorm(x, g, b):
    mu = jnp.mean(x, axis=-1, keepdims=True)
    var = jnp.mean((x - mu) ** 2, axis=-1, keepdims=True)
    return (x - mu) * lax.rsqrt(var + LN_EPS) * g + b


def _gelu(x):
    return 0.5 * x * (1.0 + lax.erf(x * 0.7071067811865476))


def _band_cat(conv_w):
    w = conv_w.reshape(LCNN, KTAP, KTAP)
    c = jnp.arange(DIM)[:, None]
    d = jnp.arange(DIM)[None, :]
    dj = c - d + WIN
    valid = (dj >= 0) & (dj < KTAP)
    djc = jnp.clip(dj, 0, KTAP - 1)
    band = jnp.where(valid[None, None], w[:, :, djc], 0.0)
    return band.transpose(0, 2, 1, 3).reshape(LCNN, DIM, KTAP * DIM)


def _shift_sum(g, bb):
    acc = None
    for di in range(KTAP):
        sl = g[:, :, di * DIM:(di + 1) * DIM]
        sh = di - WIN
        if sh > 0:
            z = jnp.zeros((bb, sh, DIM), jnp.float32)
            t = jnp.concatenate([sl[:, sh:, :], z], axis=1)
        elif sh < 0:
            z = jnp.zeros((bb, -sh, DIM), jnp.float32)
            t = jnp.concatenate([z, sl[:, :SEQ + sh, :]], axis=1)
        else:
            t = sl
        acc = t if acc is None else acc + t
    return acc


def _dti_block_kernel(phar_ref, mol_ref, prot_ref, t1_ref, band_ref, mats_ref,
                      vec_ref, wout_ref, hvec_ref, out_ref):
    bb = phar_ref.shape[0]
    f32 = jnp.float32

    idx = prot_ref[...]
    iota = lax.broadcasted_iota(jnp.int32, (bb, SEQ, NWORD), 2)
    onehot = (idx[:, :, None] == iota).astype(f32)
    g = jnp.dot(onehot.reshape(bb * SEQ, NWORD), t1_ref[...],
                preferred_element_type=f32).reshape(bb, SEQ, KTAP * DIM)
    xs = jnp.maximum(_shift_sum(g, bb) + vec_ref[5:6, :], 0.0)
    for l in range(1, LCNN):
        g = jnp.dot(xs.reshape(bb * SEQ, DIM), band_ref[l - 1],
                    preferred_element_type=f32).reshape(bb, SEQ, KTAP * DIM)
        xs = jnp.maximum(_shift_sum(g, bb) + vec_ref[5 + l:6 + l, :], 0.0)

    p = phar_ref[...]
    h1 = _gelu(jnp.dot(p, mats_ref[0:NQ * DIM, :],
                       preferred_element_type=f32) + vec_ref[0:1, :])
    prompt = jnp.dot(h1, mats_ref[NQ * DIM:NQ * DIM + DIM, :],
                     preferred_element_type=f32) + vec_ref[1:2, :]
    mol = _layernorm(prompt + mol_ref[...], vec_ref[2:3, :], vec_ref[3:4, :])

    wa = mats_ref[NQ * DIM + DIM:NQ * DIM + 2 * DIM, :]
    ba = vec_ref[4:5, :]
    h = jnp.maximum(jnp.dot(mol, wa, preferred_element_type=f32) + ba, 0.0)
    hs = jnp.maximum(
        jnp.dot(xs.reshape(bb * SEQ, DIM), wa,
                preferred_element_type=f32) + ba, 0.0).reshape(bb, SEQ, DIM)
    wts = jnp.tanh(jnp.sum(h[:, None, :] * hs, axis=-1, keepdims=True))
    prot_vec = jnp.mean(wts * hs, axis=1)

    cat = jnp.maximum(
        jnp.dot(mol, wout_ref[0:DIM, :], preferred_element_type=f32)
        + jnp.dot(prot_vec, wout_ref[DIM:2 * DIM, :],
                  preferred_element_type=f32)
        + hvec_ref[0:1, :], 0.0)
    for j in range(1, LOUT):
        wj = wout_ref[j * 2 * DIM:(j + 1) * 2 * DIM, :]
        cat = jnp.maximum(
            jnp.dot(cat, wj, preferred_element_type=f32)
            + hvec_ref[j:j + 1, :], 0.0)

    out = (jnp.sum(cat * hvec_ref[LOUT:LOUT + 1, :], axis=-1, keepdims=True)
           + hvec_ref[LOUT + 1:LOUT + 2, 0:1])
    out_ref[...] = out


@jax.jit
def _forward(phar_prompt, mol_repr, protein_batch, proj_w1, proj_b1, proj_w2,
             proj_b2, emb, mol_gamma, mol_beta, prot_gamma, prot_beta, conv_w,
             conv_b, wa, ba, wout_w, wout_b, wint_w, wint_b):
    bn = mol_repr.shape[0]
    bb = math.gcd(bn, 512)

    phar2 = phar_prompt.reshape(bn, NQ * DIM)

    band = _band_cat(conv_w)
    emb_ln = _layernorm(emb, prot_gamma, prot_beta)
    t1 = jnp.dot(emb_ln, band[0])

    mats = jnp.concatenate([proj_w1, proj_w2, wa], axis=0)
    vec = jnp.concatenate([
        proj_b1, proj_b2, mol_gamma, mol_beta, ba,
        jnp.broadcast_to(conv_b, (LCNN, DIM)),
    ], axis=0)
    wout = wout_w.reshape(LOUT * 2 * DIM, 2 * DIM)
    hvec = jnp.concatenate([
        wout_b.reshape(LOUT, 2 * DIM),
        wint_w.T,
        jnp.pad(wint_b, ((0, 0), (0, 2 * DIM - 1))),
    ], axis=0)

    out = pl.pallas_call(
        _dti_block_kernel,
        out_shape=jax.ShapeDtypeStruct((bn, 1), jnp.float32),
        grid=(bn // bb,),
        in_specs=[
            pl.BlockSpec((bb, NQ * DIM), lambda b: (b, 0)),
            pl.BlockSpec((bb, DIM), lambda b: (b, 0)),
            pl.BlockSpec((bb, SEQ), lambda b: (b, 0)),
            pl.BlockSpec((NWORD, KTAP * DIM), lambda b: (0, 0)),
            pl.BlockSpec((LCNN - 1, DIM, KTAP * DIM), lambda b: (0, 0, 0)),
            pl.BlockSpec(((NQ + 2) * DIM, DIM), lambda b: (0, 0)),
            pl.BlockSpec((5 + LCNN, DIM), lambda b: (0, 0)),
            pl.BlockSpec((LOUT * 2 * DIM, 2 * DIM), lambda b: (0, 0)),
            pl.BlockSpec((LOUT + 2, 2 * DIM), lambda b: (0, 0)),
        ],
        out_specs=pl.BlockSpec((bb, 1), lambda b: (b, 0)),
        compiler_params=pltpu.CompilerParams(
            dimension_semantics=("parallel",)),
    )(phar2, mol_repr, protein_batch, t1, band[1:], mats, vec, wout, hvec)
    return out


def kernel(phar_prompt, mol_repr, protein_batch, proj_w1, proj_b1, proj_w2,
           proj_b2, emb, mol_gamma, mol_beta, prot_gamma, prot_beta, conv_w,
           conv_b, wa, ba, wout_w, wout_b, wint_w, wint_b):
    return _forward(phar_prompt, mol_repr, protein_batch, proj_w1, proj_b1,
                    proj_w2, proj_b2, emb, mol_gamma, mol_beta, prot_gamma,
                    prot_beta, conv_w, conv_b, wa, ba, wout_w, wout_b,
                    wint_w, wint_b)


# speedup vs baseline: 195.8333x; 3.9933x over previous
import math

import jax
import jax.numpy as jnp
import numpy as np
from jax import lax
from jax.experimental import pallas as pl
from jax.experimental.pallas import tpu as pltpu

SEQ = 16
DIM = 32
NQ = 3
NWORD = 32
WIN = 2
KTAP = 2 * WIN + 1
LCNN = 3
LOUT = 3
LN_EPS = 1e-5


def _layernorm(x, g, b):
    mu = jnp.mean(x, axis=-1, keepdims=True)
    var = jnp.mean((x - mu) ** 2, axis=-1, keepdims=True)
    return (x - mu) * lax.rsqrt(var + LN_EPS) * g + b


def _gelu(x):
    return 0.5 * x * (1.0 + lax.erf(x * 0.7071067811865476))


def _band_cat(conv_w):
    w = conv_w.reshape(LCNN, KTAP, KTAP)
    c = jnp.arange(DIM)[:, None]
    d = jnp.arange(DIM)[None, :]
    dj = c - d + WIN
    valid = (dj >= 0) & (dj < KTAP)
    djc = jnp.clip(dj, 0, KTAP - 1)
    band = jnp.where(valid[None, None], w[:, :, djc], 0.0)
    return band.transpose(0, 2, 1, 3).reshape(LCNN, DIM, KTAP * DIM)


def _shift_sum_t(g, bb):
    n = SEQ * bb
    acc = None
    for di in range(KTAP):
        sl = g[di * DIM:(di + 1) * DIM, :]
        sh = (di - WIN) * bb
        if sh > 0:
            z = jnp.zeros((DIM, sh), jnp.float32)
            t = jnp.concatenate([sl[:, sh:], z], axis=1)
        elif sh < 0:
            z = jnp.zeros((DIM, -sh), jnp.float32)
            t = jnp.concatenate([z, sl[:, :n + sh]], axis=1)
        else:
            t = sl
        acc = t if acc is None else acc + t
    return acc


def _dti_block_kernel(phar_ref, mol_ref, prot_ref, packt_ref, matsn_ref,
                      vec_ref, vect_ref, packh_ref, out_ref):
    bb = phar_ref.shape[0]
    n = SEQ * bb
    f32 = jnp.float32

    idx = prot_ref[0]
    iota = lax.broadcasted_iota(jnp.int32, (NWORD, n), 0)
    onehot = (idx == iota).astype(f32)
    t1t = packt_ref[0:KTAP * DIM, :]
    g = jnp.dot(t1t, onehot, preferred_element_type=f32)
    ba_col = vect_ref[:, 0:1]
    xs = jnp.maximum(_shift_sum_t(g, bb) + vect_ref[:, 1:2], 0.0)
    for l in range(1, LCNN):
        bt = packt_ref[l * KTAP * DIM:(l + 1) * KTAP * DIM, :]
        g = jnp.dot(bt, xs, preferred_element_type=f32)
        xs = jnp.maximum(_shift_sum_t(g, bb) + vect_ref[:, 1 + l:2 + l], 0.0)

    p = phar_ref[...]
    h1 = _gelu(jnp.dot(p, matsn_ref[0:NQ * DIM, :],
                       preferred_element_type=f32) + vec_ref[0:1, :])
    prompt = jnp.dot(h1, matsn_ref[NQ * DIM:NQ * DIM + DIM, :],
                     preferred_element_type=f32) + vec_ref[1:2, :]
    mol = _layernorm(prompt + mol_ref[...], vec_ref[2:3, :], vec_ref[3:4, :])
    molt = jnp.transpose(mol)

    wat = packt_ref[LCNN * KTAP * DIM:LCNN * KTAP * DIM + DIM, :]
    ht = jnp.maximum(jnp.dot(wat, molt, preferred_element_type=f32)
                     + ba_col, 0.0)
    hst = jnp.maximum(jnp.dot(wat, xs, preferred_element_type=f32)
                      + ba_col, 0.0)
    ht_tiled = jnp.concatenate([ht] * SEQ, axis=1)
    ones_d = jnp.ones((1, DIM), f32)
    sig = jnp.dot(ones_d, ht_tiled * hst, preferred_element_type=f32)
    wts = jnp.tanh(sig)
    wprod = wts * hst
    prott = wprod[:, 0:bb]
    for s in range(1, SEQ):
        prott = prott + wprod[:, s * bb:(s + 1) * bb]
    prott = prott * (1.0 / SEQ)

    D2 = 2 * DIM
    cat = jnp.maximum(
        jnp.dot(packh_ref[0:D2, 0:DIM], molt, preferred_element_type=f32)
        + jnp.dot(packh_ref[0:D2, DIM:D2], prott, preferred_element_type=f32)
        + packh_ref[LOUT * D2:LOUT * D2 + D2, 0:1], 0.0)
    for j in range(1, LOUT):
        wjt = packh_ref[j * D2:(j + 1) * D2, :]
        cat = jnp.maximum(
            jnp.dot(wjt, cat, preferred_element_type=f32)
            + packh_ref[LOUT * D2:LOUT * D2 + D2, j:j + 1], 0.0)

    ones_2d = jnp.ones((1, D2), f32)
    wint_col = packh_ref[LOUT * D2:LOUT * D2 + D2, LOUT:LOUT + 1]
    out = (jnp.dot(ones_2d, cat * wint_col, preferred_element_type=f32)
           + vec_ref[4:5, 0:1])
    out_ref[...] = out


@jax.jit
def _forward(phar_prompt, mol_repr, protein_batch, proj_w1, proj_b1, proj_w2,
             proj_b2, emb, mol_gamma, mol_beta, prot_gamma, prot_beta, conv_w,
             conv_b, wa, ba, wout_w, wout_b, wint_w, wint_b):
    bn = mol_repr.shape[0]
    bb = math.gcd(bn, 512)
    nblk = bn // bb

    phar2 = phar_prompt.reshape(bn, NQ * DIM)
    prot_flat = protein_batch.reshape(nblk, bb, SEQ).transpose(0, 2, 1) \
                             .reshape(nblk, 1, SEQ * bb)

    band = _band_cat(conv_w)
    emb_ln = _layernorm(emb, prot_gamma, prot_beta)
    t1 = jnp.dot(emb_ln, band[0])
    packt = jnp.concatenate([
        t1.T,
        band[1].T, band[2].T,
        wa.T,
    ], axis=0)

    matsn = jnp.concatenate([proj_w1, proj_w2], axis=0)
    vec = jnp.concatenate([
        proj_b1, proj_b2, mol_gamma, mol_beta,
        jnp.pad(wint_b, ((0, 0), (0, DIM - 1))),
    ], axis=0)
    vect = jnp.concatenate([
        ba.T,
        jnp.broadcast_to(conv_b[0], (DIM, 1)),
        jnp.broadcast_to(conv_b[1], (DIM, 1)),
        jnp.broadcast_to(conv_b[2], (DIM, 1)),
        jnp.zeros((DIM, 1), jnp.float32),
    ], axis=1)

    D2 = 2 * DIM
    headmats = jnp.concatenate(
        [wout_w[j].T for j in range(LOUT)], axis=0)
    bias_cols = jnp.concatenate(
        [wout_b[j].T for j in range(LOUT)] + [wint_w], axis=1)
    packh = jnp.concatenate([
        headmats,
        jnp.pad(bias_cols, ((0, 0), (0, D2 - (LOUT + 1)))),
    ], axis=0)

    out = pl.pallas_call(
        _dti_block_kernel,
        out_shape=jax.ShapeDtypeStruct((1, bn), jnp.float32),
        grid=(nblk,),
        in_specs=[
            pl.BlockSpec((bb, NQ * DIM), lambda b: (b, 0)),
            pl.BlockSpec((bb, DIM), lambda b: (b, 0)),
            pl.BlockSpec((1, 1, SEQ * bb), lambda b: (b, 0, 0)),
            pl.BlockSpec((LCNN * KTAP * DIM + DIM, DIM), lambda b: (0, 0)),
            pl.BlockSpec(((NQ + 1) * DIM, DIM), lambda b: (0, 0)),
            pl.BlockSpec((5, DIM), lambda b: (0, 0)),
            pl.BlockSpec((DIM, 5), lambda b: (0, 0)),
            pl.BlockSpec((4 * D2, D2), lambda b: (0, 0)),
        ],
        out_specs=pl.BlockSpec((1, bb), lambda b: (0, b)),
        compiler_params=pltpu.CompilerParams(
            dimension_semantics=("parallel",)),
    )(phar2, mol_repr, prot_flat, packt, matsn, vec, vect, packh)
    return out.reshape(bn, 1)


def kernel(phar_prompt, mol_repr, protein_batch, proj_w1, proj_b1, proj_w2,
           proj_b2, emb, mol_gamma, mol_beta, prot_gamma, prot_beta, conv_w,
           conv_b, wa, ba, wout_w, wout_b, wint_w, wint_b):
    return _forward(phar_prompt, mol_repr, protein_batch, proj_w1, proj_b1,
                    proj_w2, proj_b2, emb, mol_gamma, mol_beta, prot_gamma,
                    prot_beta, conv_w, conv_b, wa, ba, wout_w, wout_b,
                    wint_w, wint_b)
